# baseline (speedup 1.0000x reference)
import jax
import jax.numpy as jnp
from jax.experimental import pallas as pl
from jax.experimental.pallas import tpu as pltpu

_NCHUNK = 8


def _dma_body(x_ref, p_ref, xo_ref, po_ref, sem_x, sem_p):
    rows = x_ref.shape[0]
    chunk = rows // _NCHUNK
    copies = [
        pltpu.make_async_copy(
            x_ref.at[pl.ds(i * chunk, chunk), :],
            xo_ref.at[pl.ds(i * chunk, chunk), :],
            sem_x.at[i],
        )
        for i in range(_NCHUNK)
    ]
    pcopy = pltpu.make_async_copy(p_ref, po_ref, sem_p)
    for c in copies:
        c.start()
    pcopy.start()
    for c in copies:
        c.wait()
    pcopy.wait()


def kernel(x, positions, rel_pos_embeddings):
    B, S, D = x.shape
    xr = x.reshape(B * S, D)
    x_out, p_out = pl.pallas_call(
        _dma_body,
        in_specs=[
            pl.BlockSpec(memory_space=pl.ANY),
            pl.BlockSpec(memory_space=pl.ANY),
        ],
        out_specs=[
            pl.BlockSpec(memory_space=pl.ANY),
            pl.BlockSpec(memory_space=pl.ANY),
        ],
        out_shape=[
            jax.ShapeDtypeStruct((B * S, D), x.dtype),
            jax.ShapeDtypeStruct(positions.shape, positions.dtype),
        ],
        scratch_shapes=[
            pltpu.SemaphoreType.DMA((_NCHUNK,)),
            pltpu.SemaphoreType.DMA,
        ],
    )(xr, positions)
    return (x_out.reshape(B, S, D), p_out)


# speedup vs baseline: 47.9468x; 47.9468x over previous
import jax
import jax.numpy as jnp
from jax.experimental import pallas as pl


def _copy_body(x_ref, p_ref, xo_ref, po_ref):
    xo_ref[...] = x_ref[...]
    po_ref[...] = p_ref[...]


def kernel(x, positions, rel_pos_embeddings):
    B, S, D = x.shape
    ROWS = 1024
    grid_n = (B * S) // ROWS
    xr = x.reshape(B * S, D)
    npos = positions.size
    pr = positions.reshape(grid_n, 1, npos // grid_n)
    x_out, p_out = pl.pallas_call(
        _copy_body,
        grid=(grid_n,),
        in_specs=[
            pl.BlockSpec((ROWS, D), lambda i: (i, 0)),
            pl.BlockSpec((1, 1, npos // grid_n), lambda i: (i, 0, 0)),
        ],
        out_specs=[
            pl.BlockSpec((ROWS, D), lambda i: (i, 0)),
            pl.BlockSpec((1, 1, npos // grid_n), lambda i: (i, 0, 0)),
        ],
        out_shape=[
            jax.ShapeDtypeStruct((B * S, D), x.dtype),
            jax.ShapeDtypeStruct(pr.shape, positions.dtype),
        ],
    )(xr, pr)
    return (x_out.reshape(B, S, D), p_out.reshape(positions.shape))
